# baseline (speedup 1.0000x reference)
import functools

import jax
import jax.numpy as jnp
from jax import lax
from jax.experimental import pallas as pl
from jax.experimental.pallas import tpu as pltpu
from jax.experimental.pallas import tpu_sc as plsc

SEQ, BATCH, DIM = 8192, 4, 2048
NC, NS = 2, 16
NW = NC * NS
ROWS_PER_W = SEQ // NW
R = 2
CHUNKS = ROWS_PER_W // R
NBUF_IN = 4
NBUF_OUT = 2
OUTER = CHUNKS // NBUF_IN


def _sc_body(x_hbm, w_hbm, out_hbm, ybuf, wbuf, obuf,
             isem0, isem1, isem2, isem3, osem0, osem1):
    cid = lax.axis_index("c")
    sid = lax.axis_index("s")
    base = (cid * NS + sid) * ROWS_PER_W
    isems = (isem0, isem1, isem2, isem3)
    osems = (osem0, osem1)

    def start_in(chunk, si):
        row0 = base + chunk * R
        pltpu.async_copy(x_hbm.at[pl.ds(row0, R)], ybuf.at[si], isems[si])
        pltpu.async_copy(w_hbm.at[pl.ds(row0, R)], wbuf.at[si], isems[si])

    def wait_in(si):
        pltpu.make_async_copy(x_hbm.at[pl.ds(base, R)], ybuf.at[si], isems[si]).wait()
        pltpu.make_async_copy(w_hbm.at[pl.ds(base, R)], wbuf.at[si], isems[si]).wait()

    def start_out(chunk, so):
        row0 = base + chunk * R
        pltpu.async_copy(obuf.at[so], out_hbm.at[pl.ds(row0, R)], osems[so])

    def wait_out(so):
        pltpu.make_async_copy(obuf.at[so], out_hbm.at[pl.ds(base, R)], osems[so]).wait()

    def compute(si, so):
        @plsc.parallel_loop(0, DIM // 16, 1, unroll=16)
        def jbody(j, _si=si, _so=so):
            off = j * 16
            for r in range(R):
                wv = wbuf[_si, r, pl.ds(off, 16)]
                for b in range(BATCH):
                    obuf[_so, r, b, pl.ds(off, 16)] = (
                        ybuf[_si, r, b, pl.ds(off, 16)] + wv
                    )

    for k in range(NBUF_IN):
        start_in(k, k)

    def outer(g, carry):
        for k in range(NBUF_IN):
            c = g * NBUF_IN + k
            si = k
            so = k % NBUF_OUT
            wait_in(si)
            if k < NBUF_OUT:
                @pl.when(g >= 1)
                def _():
                    wait_out(so)
            else:
                wait_out(so)
            compute(si, so)
            start_out(c, so)

            @pl.when(g < OUTER - 1)
            def _():
                start_in(c + NBUF_IN, si)
        return carry

    lax.fori_loop(0, OUTER, outer, 0)

    wait_out(0)
    wait_out(1)


@functools.partial(
    pl.kernel,
    mesh=plsc.VectorSubcoreMesh(core_axis_name="c", subcore_axis_name="s"),
    out_type=jax.ShapeDtypeStruct((SEQ, BATCH, DIM), jnp.float32),
    scratch_types=[
        pltpu.VMEM((NBUF_IN, R, BATCH, DIM), jnp.float32),
        pltpu.VMEM((NBUF_IN, R, DIM), jnp.float32),
        pltpu.VMEM((NBUF_OUT, R, BATCH, DIM), jnp.float32),
        pltpu.SemaphoreType.DMA,
        pltpu.SemaphoreType.DMA,
        pltpu.SemaphoreType.DMA,
        pltpu.SemaphoreType.DMA,
        pltpu.SemaphoreType.DMA,
        pltpu.SemaphoreType.DMA,
    ],
)
def _sc_add(x_hbm, w_hbm, out_hbm, ybuf, wbuf, obuf,
            isem0, isem1, isem2, isem3, osem0, osem1):
    _sc_body(x_hbm, w_hbm, out_hbm, ybuf, wbuf, obuf,
             isem0, isem1, isem2, isem3, osem0, osem1)


def kernel(x, weight):
    return _sc_add(x, weight[:SEQ])


# speedup vs baseline: 1.0008x; 1.0008x over previous
import functools

import jax
import jax.numpy as jnp
from jax import lax
from jax.experimental import pallas as pl
from jax.experimental.pallas import tpu as pltpu
from jax.experimental.pallas import tpu_sc as plsc

SEQ, BATCH, DIM = 8192, 4, 2048
NC, NS = 2, 16
NW = NC * NS
ROWS_PER_W = SEQ // NW
R = 2
CHUNKS = ROWS_PER_W // R
NBUF_IN = 4
NBUF_OUT = 2
OUTER = CHUNKS // NBUF_IN


def _sc_body(x_hbm, w_hbm, out_hbm, ybuf, wbuf, obuf,
             isem0, isem1, isem2, isem3, osem0, osem1):
    cid = lax.axis_index("c")
    sid = lax.axis_index("s")
    base = (cid * NS + sid) * ROWS_PER_W
    isems = (isem0, isem1, isem2, isem3)
    osems = (osem0, osem1)

    def start_in(chunk, si):
        row0 = base + chunk * R
        pltpu.async_copy(x_hbm.at[pl.ds(row0, R)], ybuf.at[si], isems[si])
        pltpu.async_copy(w_hbm.at[pl.ds(row0, R)], wbuf.at[si], isems[si])

    def wait_in(si):
        pltpu.make_async_copy(x_hbm.at[pl.ds(base, R)], ybuf.at[si], isems[si]).wait()
        pltpu.make_async_copy(w_hbm.at[pl.ds(base, R)], wbuf.at[si], isems[si]).wait()

    def start_out(chunk, so):
        row0 = base + chunk * R
        pltpu.async_copy(obuf.at[so], out_hbm.at[pl.ds(row0, R)], osems[so])

    def wait_out(so):
        pltpu.make_async_copy(obuf.at[so], out_hbm.at[pl.ds(base, R)], osems[so]).wait()

    def compute(si, so):
        @plsc.parallel_loop(0, DIM // 16, 1, unroll=16)
        def jbody(j, _si=si, _so=so):
            off = j * 16
            for r in range(R):
                wv = wbuf[_si, r, pl.ds(off, 16)]
                for b in range(BATCH):
                    obuf[_so, r, b, pl.ds(off, 16)] = (
                        ybuf[_si, r, b, pl.ds(off, 16)] + wv
                    )

    for k in range(NBUF_IN):
        start_in(k, k)

    def outer(g, carry):
        for k in range(NBUF_IN):
            c = g * NBUF_IN + k
            si = k
            so = k % NBUF_OUT
            if k < NBUF_OUT:
                @pl.when(g >= 1)
                def _():
                    wait_out(so)
            else:
                wait_out(so)
            wait_in(si)
            compute(si, so)
            start_out(c, so)

            @pl.when(g < OUTER - 1)
            def _():
                start_in(c + NBUF_IN, si)
        return carry

    lax.fori_loop(0, OUTER, outer, 0)

    wait_out(0)
    wait_out(1)


@functools.partial(
    pl.kernel,
    mesh=plsc.VectorSubcoreMesh(core_axis_name="c", subcore_axis_name="s"),
    out_type=jax.ShapeDtypeStruct((SEQ, BATCH, DIM), jnp.float32),
    scratch_types=[
        pltpu.VMEM((NBUF_IN, R, BATCH, DIM), jnp.float32),
        pltpu.VMEM((NBUF_IN, R, DIM), jnp.float32),
        pltpu.VMEM((NBUF_OUT, R, BATCH, DIM), jnp.float32),
        pltpu.SemaphoreType.DMA,
        pltpu.SemaphoreType.DMA,
        pltpu.SemaphoreType.DMA,
        pltpu.SemaphoreType.DMA,
        pltpu.SemaphoreType.DMA,
        pltpu.SemaphoreType.DMA,
    ],
)
def _sc_add(x_hbm, w_hbm, out_hbm, ybuf, wbuf, obuf,
            isem0, isem1, isem2, isem3, osem0, osem1):
    _sc_body(x_hbm, w_hbm, out_hbm, ybuf, wbuf, obuf,
             isem0, isem1, isem2, isem3, osem0, osem1)


def kernel(x, weight):
    return _sc_add(x, weight[:SEQ])
